# Initial kernel scaffold; baseline (speedup 1.0000x reference)
import jax
import jax.numpy as jnp
from jax.experimental import pallas as pl


def kernel(x, edge_index, batch, W1, b1, g1, be1, W2, b2, g2, be2, W3, b3, g3, be3, Wout, bout):
    raise NotImplementedError("write your pallas kernel here")


# speedup vs baseline: 8.7519x; 8.7519x over previous
import functools

import jax
import jax.numpy as jnp
from jax import lax
from jax.experimental import pallas as pl
from jax.experimental.pallas import tpu as pltpu
from jax.experimental.pallas import tpu_sc as plsc

N = 10000
E = 320000
D = 128
H = 128
G = 32
T = 5

NC = 2
NS = 16
NP = 10112
RPS = NP // NS
EC = 128
CPS = 80
EPAD = NC * NS * CPS * EC
KF = RPS // EC
KT = RPS % EC

_mesh = plsc.VectorSubcoreMesh(core_axis_name="c", subcore_axis_name="s")


@jax.jit
def _sc_hist(idxm):
    @functools.partial(
        pl.kernel,
        out_type=jax.ShapeDtypeStruct((NC, NP, 16), jnp.float32),
        mesh=_mesh,
        scratch_types=[
            pltpu.VMEM((CPS, 2, EC), jnp.int32),
            pltpu.VMEM((EC, 16), jnp.float32),
            pltpu.VMEM_SHARED((NP, 16), jnp.float32),
        ],
    )
    def k(idx_hbm, out_hbm, idx_v, buf, acc):
        c = lax.axis_index("c")
        s = lax.axis_index("s")
        pltpu.sync_copy(idx_hbm.at[c].at[s], idx_v)

        zero16 = jnp.zeros((16,), jnp.float32)

        @pl.loop(0, EC)
        def _(i):
            buf[i, pl.ds(0, 16)] = zero16

        base = pl.multiple_of(s * RPS, 8)

        @pl.loop(0, KF)
        def _(kk):
            pltpu.sync_copy(buf, acc.at[pl.ds(base + kk * EC, EC)])

        pltpu.sync_copy(buf.at[pl.ds(0, KT)],
                        acc.at[pl.ds(base + KF * EC, KT)])

        one16 = jnp.ones((16,), jnp.float32)

        @pl.loop(0, EC)
        def _(i):
            buf[i, pl.ds(0, 16)] = one16

        plsc.subcore_barrier()

        @pl.loop(0, CPS)
        def _(j):
            pltpu.sync_copy(buf, acc.at[idx_v.at[j].at[1]], add=True)

        plsc.subcore_barrier()
        pltpu.sync_copy(acc.at[pl.ds(base, RPS)],
                        out_hbm.at[c].at[pl.ds(base, RPS)])

    return k(idxm)


@jax.jit
def _sc_scatter(u, idxm):
    @functools.partial(
        pl.kernel,
        out_type=jax.ShapeDtypeStruct((NC, NP, H), jnp.float32),
        mesh=_mesh,
        scratch_types=[
            pltpu.VMEM((2, EC), jnp.int32),
            pltpu.VMEM((2, EC), jnp.int32),
            pltpu.VMEM((EC, H), jnp.float32),
            pltpu.VMEM((EC, H), jnp.float32),
            pltpu.VMEM_SHARED((NP, H), jnp.float32),
            pltpu.SemaphoreType.DMA,
            pltpu.SemaphoreType.DMA,
            pltpu.SemaphoreType.DMA,
            pltpu.SemaphoreType.DMA,
        ],
    )
    def k(u_hbm, idx_hbm, out_hbm, ring0, ring1, buf0, buf1, acc,
          isem0, isem1, gsem0, gsem1):
        c = lax.axis_index("c")
        s = lax.axis_index("s")
        idxc = idx_hbm.at[c].at[s]

        zero16 = jnp.zeros((16,), jnp.float32)

        @pl.loop(0, EC)
        def _(i):
            @pl.loop(0, H, step=16)
            def _(jj):
                buf0[i, pl.ds(jj, 16)] = zero16

        base = pl.multiple_of(s * RPS, 8)

        @pl.loop(0, KF)
        def _(kk):
            pltpu.sync_copy(buf0, acc.at[pl.ds(base + kk * EC, EC)])

        pltpu.sync_copy(buf0.at[pl.ds(0, KT)],
                        acc.at[pl.ds(base + KF * EC, KT)])
        plsc.subcore_barrier()

        pltpu.async_copy(idxc.at[0], ring0, isem0)
        pltpu.async_copy(idxc.at[1], ring1, isem1)
        pltpu.make_async_copy(idxc.at[0], ring0, isem0).wait()
        pltpu.async_copy(u_hbm.at[ring0.at[0]], buf0, gsem0)

        @pl.loop(0, CPS - 2, step=2)
        def _(j):
            pltpu.make_async_copy(idxc.at[j + 1], ring1, isem1).wait()
            pltpu.async_copy(u_hbm.at[ring1.at[0]], buf1, gsem1)
            pltpu.make_async_copy(u_hbm.at[ring0.at[0]], buf0, gsem0).wait()
            pltpu.sync_copy(buf0, acc.at[ring0.at[1]], add=True)
            pltpu.async_copy(idxc.at[j + 2], ring0, isem0)
            pltpu.make_async_copy(idxc.at[j + 2], ring0, isem0).wait()
            pltpu.async_copy(u_hbm.at[ring0.at[0]], buf0, gsem0)
            pltpu.make_async_copy(u_hbm.at[ring1.at[0]], buf1, gsem1).wait()
            pltpu.sync_copy(buf1, acc.at[ring1.at[1]], add=True)
            pltpu.async_copy(idxc.at[j + 3], ring1, isem1)

        pltpu.make_async_copy(idxc.at[CPS - 1], ring1, isem1).wait()
        pltpu.async_copy(u_hbm.at[ring1.at[0]], buf1, gsem1)
        pltpu.make_async_copy(u_hbm.at[ring0.at[0]], buf0, gsem0).wait()
        pltpu.sync_copy(buf0, acc.at[ring0.at[1]], add=True)
        pltpu.make_async_copy(u_hbm.at[ring1.at[0]], buf1, gsem1).wait()
        pltpu.sync_copy(buf1, acc.at[ring1.at[1]], add=True)

        plsc.subcore_barrier()
        pltpu.sync_copy(acc.at[pl.ds(base, RPS)],
                        out_hbm.at[c].at[pl.ds(base, RPS)])

    return k(u, idxm)


def _valid_mask():
    rows = lax.broadcasted_iota(jnp.int32, (NP, H), 0)
    return rows < N


def _tc_pre(x, W1, hist):
    def body(x_ref, w_ref, hist_ref, u_ref, dinv_ref):
        xw = jnp.dot(x_ref[...], w_ref[...], preferred_element_type=jnp.float32)
        cnt = hist_ref[0, :, 0:1] + hist_ref[1, :, 0:1]
        deg = jnp.broadcast_to(cnt + 1.0, (NP, H))
        dinv = jnp.where(_valid_mask(), lax.rsqrt(deg), 0.0)
        dinv_ref[...] = dinv
        u_ref[:N, :] = xw * dinv[:N, :]
        u_ref[N:, :] = jnp.zeros((NP - N, H), jnp.float32)

    return pl.pallas_call(
        body,
        out_shape=(jax.ShapeDtypeStruct((NP, H), jnp.float32),
                   jax.ShapeDtypeStruct((NP, H), jnp.float32)),
    )(x, W1, hist)


def _bn_relu(y_ref, u_ref, dinv, b_ref, g_ref, be_ref):
    z = dinv * (y_ref[0] + y_ref[1] + u_ref[...]) + b_ref[...][None, :]
    zm = jnp.where(_valid_mask(), z, 0.0)
    s1 = jnp.sum(zm, axis=0)
    s2 = jnp.sum(zm * zm, axis=0)
    m = s1 / N
    v = s2 / N - m * m
    scale = lax.rsqrt(v + 1e-5) * g_ref[...]
    return jnp.maximum((z - m[None, :]) * scale[None, :] + be_ref[...][None, :],
                       0.0)


def _tc_mid(y, u, dinv, b, g, be, Wn):
    def body(y_ref, u_ref, dinv_ref, b_ref, g_ref, be_ref, w_ref, out_ref):
        dinv = dinv_ref[...]
        hn = _bn_relu(y_ref, u_ref, dinv, b_ref, g_ref, be_ref)
        out_ref[...] = dinv * jnp.dot(hn, w_ref[...],
                                      preferred_element_type=jnp.float32)

    return pl.pallas_call(
        body,
        out_shape=jax.ShapeDtypeStruct((NP, H), jnp.float32),
    )(y, u, dinv, b, g, be, Wn)


def _tc_post(y, u, dinv, b, g, be, batch_p, Wout, bout):
    def body(y_ref, u_ref, dinv_ref, b_ref, g_ref, be_ref, batch_ref,
             wout_ref, bout_ref, out_ref):
        hn = _bn_relu(y_ref, u_ref, dinv_ref[...], b_ref, g_ref, be_ref)
        gids = lax.broadcasted_iota(jnp.int32, (G, NP), 0)
        onehot = (batch_ref[...][None, :] == gids).astype(jnp.float32)
        cnt = jnp.sum(onehot, axis=1)
        pooled = jnp.dot(onehot, hn, preferred_element_type=jnp.float32)
        pooled = pooled / jnp.maximum(cnt, 1.0)[:, None]
        out_ref[...] = (jnp.dot(pooled, wout_ref[...],
                                preferred_element_type=jnp.float32)
                        + bout_ref[...][None, :])

    return pl.pallas_call(
        body,
        out_shape=jax.ShapeDtypeStruct((G, T), jnp.float32),
    )(y, u, dinv, b, g, be, batch_p, Wout, bout)


def kernel(x, edge_index, batch, W1, b1, g1, be1, W2, b2, g2, be2,
           W3, b3, g3, be3, Wout, bout):
    src = edge_index[0]
    dst = edge_index[1]
    padv = jnp.full((EPAD - E,), N, jnp.int32)
    srcm = jnp.concatenate([src, padv]).reshape(NC, NS, CPS, 1, EC)
    dstm = jnp.concatenate([dst, padv]).reshape(NC, NS, CPS, 1, EC)
    idxm = jnp.concatenate([srcm, dstm], axis=3)
    batch_p = jnp.concatenate([batch, jnp.full((NP - N,), G, jnp.int32)])

    hist = _sc_hist(idxm)
    u1, dinv = _tc_pre(x, W1, hist)
    y1 = _sc_scatter(u1, idxm)
    u2 = _tc_mid(y1, u1, dinv, b1, g1, be1, W2)
    y2 = _sc_scatter(u2, idxm)
    u3 = _tc_mid(y2, u2, dinv, b2, g2, be2, W3)
    y3 = _sc_scatter(u3, idxm)
    return _tc_post(y3, u3, dinv, b3, g3, be3, batch_p, Wout, bout)


# speedup vs baseline: 26.3570x; 3.0116x over previous
import functools

import jax
import jax.numpy as jnp
from jax import lax
from jax.experimental import pallas as pl
from jax.experimental.pallas import tpu as pltpu
from jax.experimental.pallas import tpu_sc as plsc

N = 10000
E = 320000
D = 128
H = 128
G = 32
T = 5

NC = 2
NS = 16
NP = 10112
RPS = NP // NS
EC = 128
CPS = 80
EPAD = NC * NS * CPS * EC
KF = RPS // EC
KT = RPS % EC

_mesh = plsc.VectorSubcoreMesh(core_axis_name="c", subcore_axis_name="s")


@jax.jit
def _sc_hist(idxm):
    @functools.partial(
        pl.kernel,
        out_type=jax.ShapeDtypeStruct((NC, NP, 16), jnp.float32),
        mesh=_mesh,
        scratch_types=[
            pltpu.VMEM((CPS, 2, EC), jnp.int32),
            pltpu.VMEM((EC, 16), jnp.float32),
            pltpu.VMEM_SHARED((NP, 16), jnp.float32),
        ],
    )
    def k(idx_hbm, out_hbm, idx_v, buf, acc):
        c = lax.axis_index("c")
        s = lax.axis_index("s")
        pltpu.sync_copy(idx_hbm.at[c].at[s], idx_v)

        zero16 = jnp.zeros((16,), jnp.float32)

        @pl.loop(0, EC)
        def _(i):
            buf[i, pl.ds(0, 16)] = zero16

        base = pl.multiple_of(s * RPS, 8)

        @pl.loop(0, KF)
        def _(kk):
            pltpu.sync_copy(buf, acc.at[pl.ds(base + kk * EC, EC)])

        pltpu.sync_copy(buf.at[pl.ds(0, KT)],
                        acc.at[pl.ds(base + KF * EC, KT)])

        one16 = jnp.ones((16,), jnp.float32)

        @pl.loop(0, EC)
        def _(i):
            buf[i, pl.ds(0, 16)] = one16

        plsc.subcore_barrier()

        @pl.loop(0, CPS)
        def _(j):
            pltpu.sync_copy(buf, acc.at[idx_v.at[j].at[1]], add=True)

        plsc.subcore_barrier()
        pltpu.sync_copy(acc.at[pl.ds(base, RPS)],
                        out_hbm.at[c].at[pl.ds(base, RPS)])

    return k(idxm)


@jax.jit
def _sc_scatter(u, idxm):
    @functools.partial(
        pl.kernel,
        out_type=jax.ShapeDtypeStruct((NC, NP, H), jnp.float32),
        mesh=_mesh,
        scratch_types=[
            pltpu.VMEM((2, EC), jnp.int32),
            pltpu.VMEM((2, EC), jnp.int32),
            pltpu.VMEM((EC, H), jnp.float32),
            pltpu.VMEM((EC, H), jnp.float32),
            pltpu.VMEM_SHARED((NP, H), jnp.float32),
            pltpu.SemaphoreType.DMA,
            pltpu.SemaphoreType.DMA,
            pltpu.SemaphoreType.DMA,
            pltpu.SemaphoreType.DMA,
        ],
    )
    def k(u_hbm, idx_hbm, out_hbm, ring0, ring1, buf0, buf1, acc,
          isem0, isem1, gsem0, gsem1):
        c = lax.axis_index("c")
        s = lax.axis_index("s")
        idxc = idx_hbm.at[c].at[s]

        zero16 = jnp.zeros((16,), jnp.float32)

        @pl.loop(0, EC)
        def _(i):
            @pl.loop(0, H, step=16)
            def _(jj):
                buf0[i, pl.ds(jj, 16)] = zero16

        base = pl.multiple_of(s * RPS, 8)

        @pl.loop(0, KF)
        def _(kk):
            pltpu.sync_copy(buf0, acc.at[pl.ds(base + kk * EC, EC)])

        pltpu.sync_copy(buf0.at[pl.ds(0, KT)],
                        acc.at[pl.ds(base + KF * EC, KT)])
        plsc.subcore_barrier()

        pltpu.async_copy(idxc.at[0], ring0, isem0)
        pltpu.async_copy(idxc.at[1], ring1, isem1)
        pltpu.make_async_copy(idxc.at[0], ring0, isem0).wait()
        pltpu.async_copy(u_hbm.at[ring0.at[0]], buf0, gsem0)

        @pl.loop(0, CPS - 2, step=2)
        def _(j):
            pltpu.make_async_copy(idxc.at[j + 1], ring1, isem1).wait()
            pltpu.async_copy(u_hbm.at[ring1.at[0]], buf1, gsem1)
            pltpu.make_async_copy(u_hbm.at[ring0.at[0]], buf0, gsem0).wait()
            pltpu.sync_copy(buf0, acc.at[ring0.at[1]], add=True)
            pltpu.async_copy(idxc.at[j + 2], ring0, isem0)
            pltpu.make_async_copy(idxc.at[j + 2], ring0, isem0).wait()
            pltpu.async_copy(u_hbm.at[ring0.at[0]], buf0, gsem0)
            pltpu.make_async_copy(u_hbm.at[ring1.at[0]], buf1, gsem1).wait()
            pltpu.sync_copy(buf1, acc.at[ring1.at[1]], add=True)
            pltpu.async_copy(idxc.at[j + 3], ring1, isem1)

        pltpu.make_async_copy(idxc.at[CPS - 1], ring1, isem1).wait()
        pltpu.async_copy(u_hbm.at[ring1.at[0]], buf1, gsem1)
        pltpu.make_async_copy(u_hbm.at[ring0.at[0]], buf0, gsem0).wait()
        pltpu.sync_copy(buf0, acc.at[ring0.at[1]], add=True)
        pltpu.make_async_copy(u_hbm.at[ring1.at[0]], buf1, gsem1).wait()
        pltpu.sync_copy(buf1, acc.at[ring1.at[1]], add=True)

        plsc.subcore_barrier()
        pltpu.sync_copy(acc.at[pl.ds(base, RPS)],
                        out_hbm.at[c].at[pl.ds(base, RPS)])

    return k(u, idxm)


def _valid_mask():
    rows = lax.broadcasted_iota(jnp.int32, (NP, H), 0)
    return rows < N


def _tc_pre(x, W1, hist):
    def body(x_ref, w_ref, hist_ref, u_ref, dinv_ref):
        xw = jnp.dot(x_ref[...], w_ref[...], preferred_element_type=jnp.float32)
        cnt = hist_ref[0, :, 0:1] + hist_ref[1, :, 0:1]
        deg = jnp.broadcast_to(cnt + 1.0, (NP, H))
        dinv = jnp.where(_valid_mask(), lax.rsqrt(deg), 0.0)
        dinv_ref[...] = dinv
        u_ref[:N, :] = xw * dinv[:N, :]
        u_ref[N:, :] = jnp.zeros((NP - N, H), jnp.float32)

    return pl.pallas_call(
        body,
        out_shape=(jax.ShapeDtypeStruct((NP, H), jnp.float32),
                   jax.ShapeDtypeStruct((NP, H), jnp.float32)),
    )(x, W1, hist)


def _bn_relu(y_ref, u_ref, dinv, b_ref, g_ref, be_ref):
    z = dinv * (y_ref[0] + y_ref[1] + u_ref[...]) + b_ref[...][None, :]
    zm = jnp.where(_valid_mask(), z, 0.0)
    s1 = jnp.sum(zm, axis=0)
    s2 = jnp.sum(zm * zm, axis=0)
    m = s1 / N
    v = s2 / N - m * m
    scale = lax.rsqrt(v + 1e-5) * g_ref[...]
    return jnp.maximum((z - m[None, :]) * scale[None, :] + be_ref[...][None, :],
                       0.0)


def _tc_mid(y, u, dinv, b, g, be, Wn):
    def body(y_ref, u_ref, dinv_ref, b_ref, g_ref, be_ref, w_ref, out_ref):
        dinv = dinv_ref[...]
        hn = _bn_relu(y_ref, u_ref, dinv, b_ref, g_ref, be_ref)
        out_ref[...] = dinv * jnp.dot(hn, w_ref[...],
                                      preferred_element_type=jnp.float32)

    return pl.pallas_call(
        body,
        out_shape=jax.ShapeDtypeStruct((NP, H), jnp.float32),
    )(y, u, dinv, b, g, be, Wn)


def _tc_post(y, u, dinv, b, g, be, batch_p, Wout, bout):
    def body(y_ref, u_ref, dinv_ref, b_ref, g_ref, be_ref, batch_ref,
             wout_ref, bout_ref, out_ref):
        hn = _bn_relu(y_ref, u_ref, dinv_ref[...], b_ref, g_ref, be_ref)
        gids = lax.broadcasted_iota(jnp.int32, (G, NP), 0)
        onehot = (batch_ref[...][None, :] == gids).astype(jnp.float32)
        cnt = jnp.sum(onehot, axis=1)
        pooled = jnp.dot(onehot, hn, preferred_element_type=jnp.float32)
        pooled = pooled / jnp.maximum(cnt, 1.0)[:, None]
        out_ref[...] = (jnp.dot(pooled, wout_ref[...],
                                preferred_element_type=jnp.float32)
                        + bout_ref[...][None, :])

    return pl.pallas_call(
        body,
        out_shape=jax.ShapeDtypeStruct((G, T), jnp.float32),
    )(y, u, dinv, b, g, be, batch_p, Wout, bout)


def kernel(x, edge_index, batch, W1, b1, g1, be1, W2, b2, g2, be2,
           W3, b3, g3, be3, Wout, bout):
    src = edge_index[0]
    dst = edge_index[1]
    padv = N + jnp.arange(EPAD - E, dtype=jnp.int32) % (NP - N)
    srcm = jnp.concatenate([src, padv]).reshape(NC, NS, CPS, 1, EC)
    dstm = jnp.concatenate([dst, padv]).reshape(NC, NS, CPS, 1, EC)
    idxm = jnp.concatenate([srcm, dstm], axis=3)
    batch_p = jnp.concatenate([batch, jnp.full((NP - N,), G, jnp.int32)])

    hist = _sc_hist(idxm)
    u1, dinv = _tc_pre(x, W1, hist)
    y1 = _sc_scatter(u1, idxm)
    u2 = _tc_mid(y1, u1, dinv, b1, g1, be1, W2)
    y2 = _sc_scatter(u2, idxm)
    u3 = _tc_mid(y2, u2, dinv, b2, g2, be2, W3)
    y3 = _sc_scatter(u3, idxm)
    return _tc_post(y3, u3, dinv, b3, g3, be3, batch_p, Wout, bout)


# speedup vs baseline: 29.3152x; 1.1122x over previous
import functools

import jax
import jax.numpy as jnp
from jax import lax
from jax.experimental import pallas as pl
from jax.experimental.pallas import tpu as pltpu
from jax.experimental.pallas import tpu_sc as plsc

N = 10000
E = 320000
D = 128
H = 128
G = 32
T = 5

NC = 2
NS = 16
NP = 10112
RPS = NP // NS
EC = 128
CPS = 80
EPAD = NC * NS * CPS * EC
CB = 20
NB = CPS // CB
KF = RPS // EC
KT = RPS % EC

_mesh = plsc.VectorSubcoreMesh(core_axis_name="c", subcore_axis_name="s")


@jax.jit
def _sc_hist(idxm):
    @functools.partial(
        pl.kernel,
        out_type=jax.ShapeDtypeStruct((NC, NP, 16), jnp.float32),
        mesh=_mesh,
        scratch_types=[
            pltpu.VMEM((CPS, 2, EC), jnp.int32),
            pltpu.VMEM((EC, 16), jnp.float32),
            pltpu.VMEM_SHARED((NP, 16), jnp.float32),
        ],
    )
    def k(idx_hbm, out_hbm, idx_v, buf, acc):
        c = lax.axis_index("c")
        s = lax.axis_index("s")
        pltpu.sync_copy(idx_hbm.at[c].at[s], idx_v)

        zero16 = jnp.zeros((16,), jnp.float32)

        @pl.loop(0, EC)
        def _(i):
            buf[i, pl.ds(0, 16)] = zero16

        base = pl.multiple_of(s * RPS, 8)

        @pl.loop(0, KF)
        def _(kk):
            pltpu.sync_copy(buf, acc.at[pl.ds(base + kk * EC, EC)])

        pltpu.sync_copy(buf.at[pl.ds(0, KT)],
                        acc.at[pl.ds(base + KF * EC, KT)])

        one16 = jnp.ones((16,), jnp.float32)

        @pl.loop(0, EC)
        def _(i):
            buf[i, pl.ds(0, 16)] = one16

        plsc.subcore_barrier()

        @pl.loop(0, CPS)
        def _(j):
            pltpu.sync_copy(buf, acc.at[idx_v.at[j].at[1]], add=True)

        plsc.subcore_barrier()
        pltpu.sync_copy(acc.at[pl.ds(base, RPS)],
                        out_hbm.at[c].at[pl.ds(base, RPS)])

    return k(idxm)


@jax.jit
def _sc_scatter(u, idxm):
    @functools.partial(
        pl.kernel,
        out_type=jax.ShapeDtypeStruct((NC, NP, H), jnp.float32),
        mesh=_mesh,
        scratch_types=[
            pltpu.VMEM((CB, 2, EC), jnp.int32),
            pltpu.VMEM((CB, 2, EC), jnp.int32),
            pltpu.VMEM((EC, H), jnp.float32),
            pltpu.VMEM((EC, H), jnp.float32),
            pltpu.VMEM_SHARED((NP, H), jnp.float32),
            pltpu.SemaphoreType.DMA,
            pltpu.SemaphoreType.DMA,
            pltpu.SemaphoreType.DMA,
            pltpu.SemaphoreType.DMA,
        ],
    )
    def k(u_hbm, idx_hbm, out_hbm, ib0, ib1, buf0, buf1, acc,
          isem0, isem1, gsem0, gsem1):
        c = lax.axis_index("c")
        s = lax.axis_index("s")
        idxc = idx_hbm.at[c].at[s]
        ibs = (ib0, ib1)
        isems = (isem0, isem1)

        pltpu.async_copy(idxc.at[pl.ds(0, CB)], ib0, isem0)

        zero16 = jnp.zeros((16,), jnp.float32)

        @pl.loop(0, EC)
        def _(i):
            @pl.loop(0, H, step=16)
            def _(jj):
                buf0[i, pl.ds(jj, 16)] = zero16

        base = pl.multiple_of(s * RPS, 8)

        @pl.loop(0, KF)
        def _(kk):
            pltpu.sync_copy(buf0, acc.at[pl.ds(base + kk * EC, EC)])

        pltpu.sync_copy(buf0.at[pl.ds(0, KT)],
                        acc.at[pl.ds(base + KF * EC, KT)])
        plsc.subcore_barrier()

        pltpu.make_async_copy(idxc.at[pl.ds(0, CB)], ib0, isem0).wait()
        pltpu.async_copy(idxc.at[pl.ds(CB, CB)], ib1, isem1)
        pltpu.async_copy(u_hbm.at[ib0.at[0].at[0]], buf0, gsem0)

        for b in range(NB):
            ib = ibs[b % 2]
            ib_next = ibs[(b + 1) % 2]
            isem_next = isems[(b + 1) % 2]

            @pl.loop(0, CB - 2, step=2)
            def _(j):
                pltpu.async_copy(u_hbm.at[ib.at[j + 1].at[0]], buf1, gsem1)
                pltpu.make_async_copy(u_hbm.at[ib.at[j].at[0]], buf0,
                                      gsem0).wait()
                pltpu.sync_copy(buf0, acc.at[ib.at[j].at[1]], add=True)
                pltpu.async_copy(u_hbm.at[ib.at[j + 2].at[0]], buf0, gsem0)
                pltpu.make_async_copy(u_hbm.at[ib.at[j + 1].at[0]], buf1,
                                      gsem1).wait()
                pltpu.sync_copy(buf1, acc.at[ib.at[j + 1].at[1]], add=True)

            pltpu.async_copy(u_hbm.at[ib.at[CB - 1].at[0]], buf1, gsem1)
            pltpu.make_async_copy(u_hbm.at[ib.at[CB - 2].at[0]], buf0,
                                  gsem0).wait()
            pltpu.sync_copy(buf0, acc.at[ib.at[CB - 2].at[1]], add=True)
            if b + 1 < NB:
                pltpu.make_async_copy(idxc.at[pl.ds((b + 1) * CB, CB)],
                                      ib_next, isem_next).wait()
                pltpu.async_copy(u_hbm.at[ib_next.at[0].at[0]], buf0, gsem0)
            pltpu.make_async_copy(u_hbm.at[ib.at[CB - 1].at[0]], buf1,
                                  gsem1).wait()
            pltpu.sync_copy(buf1, acc.at[ib.at[CB - 1].at[1]], add=True)
            if b + 2 < NB:
                pltpu.async_copy(idxc.at[pl.ds((b + 2) * CB, CB)],
                                 ib, isems[b % 2])

        plsc.subcore_barrier()
        pltpu.sync_copy(acc.at[pl.ds(base, RPS)],
                        out_hbm.at[c].at[pl.ds(base, RPS)])

    return k(u, idxm)


def _valid_mask():
    rows = lax.broadcasted_iota(jnp.int32, (NP, H), 0)
    return rows < N


def _tc_pre(x, W1, hist):
    def body(x_ref, w_ref, hist_ref, u_ref, dinv_ref):
        xw = jnp.dot(x_ref[...], w_ref[...], preferred_element_type=jnp.float32)
        cnt = hist_ref[0, :, 0:1] + hist_ref[1, :, 0:1]
        deg = jnp.broadcast_to(cnt + 1.0, (NP, H))
        dinv = jnp.where(_valid_mask(), lax.rsqrt(deg), 0.0)
        dinv_ref[...] = dinv
        u_ref[:N, :] = xw * dinv[:N, :]
        u_ref[N:, :] = jnp.zeros((NP - N, H), jnp.float32)

    return pl.pallas_call(
        body,
        out_shape=(jax.ShapeDtypeStruct((NP, H), jnp.float32),
                   jax.ShapeDtypeStruct((NP, H), jnp.float32)),
    )(x, W1, hist)


def _bn_relu(y_ref, u_ref, dinv, b_ref, g_ref, be_ref):
    z = dinv * (y_ref[0] + y_ref[1] + u_ref[...]) + b_ref[...][None, :]
    zm = jnp.where(_valid_mask(), z, 0.0)
    s1 = jnp.sum(zm, axis=0)
    s2 = jnp.sum(zm * zm, axis=0)
    m = s1 / N
    v = s2 / N - m * m
    scale = lax.rsqrt(v + 1e-5) * g_ref[...]
    return jnp.maximum((z - m[None, :]) * scale[None, :] + be_ref[...][None, :],
                       0.0)


def _tc_mid(y, u, dinv, b, g, be, Wn):
    def body(y_ref, u_ref, dinv_ref, b_ref, g_ref, be_ref, w_ref, out_ref):
        dinv = dinv_ref[...]
        hn = _bn_relu(y_ref, u_ref, dinv, b_ref, g_ref, be_ref)
        out_ref[...] = dinv * jnp.dot(hn, w_ref[...],
                                      preferred_element_type=jnp.float32)

    return pl.pallas_call(
        body,
        out_shape=jax.ShapeDtypeStruct((NP, H), jnp.float32),
    )(y, u, dinv, b, g, be, Wn)


def _tc_post(y, u, dinv, b, g, be, batch_p, Wout, bout):
    def body(y_ref, u_ref, dinv_ref, b_ref, g_ref, be_ref, batch_ref,
             wout_ref, bout_ref, out_ref):
        hn = _bn_relu(y_ref, u_ref, dinv_ref[...], b_ref, g_ref, be_ref)
        gids = lax.broadcasted_iota(jnp.int32, (G, NP), 0)
        onehot = (batch_ref[...][None, :] == gids).astype(jnp.float32)
        cnt = jnp.sum(onehot, axis=1)
        pooled = jnp.dot(onehot, hn, preferred_element_type=jnp.float32)
        pooled = pooled / jnp.maximum(cnt, 1.0)[:, None]
        out_ref[...] = (jnp.dot(pooled, wout_ref[...],
                                preferred_element_type=jnp.float32)
                        + bout_ref[...][None, :])

    return pl.pallas_call(
        body,
        out_shape=jax.ShapeDtypeStruct((G, T), jnp.float32),
    )(y, u, dinv, b, g, be, batch_p, Wout, bout)


def kernel(x, edge_index, batch, W1, b1, g1, be1, W2, b2, g2, be2,
           W3, b3, g3, be3, Wout, bout):
    src = edge_index[0]
    dst = edge_index[1]
    padv = N + jnp.arange(EPAD - E, dtype=jnp.int32) % (NP - N)
    srcm = jnp.concatenate([src, padv]).reshape(NC, NS, CPS, 1, EC)
    dstm = jnp.concatenate([dst, padv]).reshape(NC, NS, CPS, 1, EC)
    idxm = jnp.concatenate([srcm, dstm], axis=3)
    batch_p = jnp.concatenate([batch, jnp.full((NP - N,), G, jnp.int32)])

    hist = _sc_hist(idxm)
    u1, dinv = _tc_pre(x, W1, hist)
    y1 = _sc_scatter(u1, idxm)
    u2 = _tc_mid(y1, u1, dinv, b1, g1, be1, W2)
    y2 = _sc_scatter(u2, idxm)
    u3 = _tc_mid(y2, u2, dinv, b2, g2, be2, W3)
    y3 = _sc_scatter(u3, idxm)
    return _tc_post(y3, u3, dinv, b3, g3, be3, batch_p, Wout, bout)


# speedup vs baseline: 29.4187x; 1.0035x over previous
import functools

import jax
import jax.numpy as jnp
from jax import lax
from jax.experimental import pallas as pl
from jax.experimental.pallas import tpu as pltpu
from jax.experimental.pallas import tpu_sc as plsc

N = 10000
E = 320000
D = 128
H = 128
G = 32
T = 5

NC = 2
NS = 16
NP = 10112
RPS = NP // NS
EC = 128
CPS = 80
EPAD = NC * NS * CPS * EC
CB = 16
NB = CPS // CB
KF = RPS // EC
KT = RPS % EC

_mesh = plsc.VectorSubcoreMesh(core_axis_name="c", subcore_axis_name="s")


@jax.jit
def _sc_hist(dstm):
    @functools.partial(
        pl.kernel,
        out_type=jax.ShapeDtypeStruct((NC, NP, 16), jnp.float32),
        mesh=_mesh,
        scratch_types=[
            pltpu.VMEM((CPS, EC), jnp.int32),
            pltpu.VMEM((EC, 16), jnp.float32),
            pltpu.VMEM_SHARED((NP, 16), jnp.float32),
        ],
    )
    def k(dst_hbm, out_hbm, idx_v, buf, acc):
        c = lax.axis_index("c")
        s = lax.axis_index("s")
        pltpu.sync_copy(dst_hbm.at[c].at[s], idx_v)

        zero16 = jnp.zeros((16,), jnp.float32)

        @pl.loop(0, EC)
        def _(i):
            buf[i, pl.ds(0, 16)] = zero16

        base = pl.multiple_of(s * RPS, 8)

        @pl.loop(0, KF)
        def _(kk):
            pltpu.sync_copy(buf, acc.at[pl.ds(base + kk * EC, EC)])

        pltpu.sync_copy(buf.at[pl.ds(0, KT)],
                        acc.at[pl.ds(base + KF * EC, KT)])

        one16 = jnp.ones((16,), jnp.float32)

        @pl.loop(0, EC)
        def _(i):
            buf[i, pl.ds(0, 16)] = one16

        plsc.subcore_barrier()

        @pl.loop(0, CPS)
        def _(j):
            pltpu.sync_copy(buf, acc.at[idx_v.at[j]], add=True)

        plsc.subcore_barrier()
        pltpu.sync_copy(acc.at[pl.ds(base, RPS)],
                        out_hbm.at[c].at[pl.ds(base, RPS)])

    return k(dstm)


@jax.jit
def _sc_scatter(u, srcm, dstm):
    @functools.partial(
        pl.kernel,
        out_type=jax.ShapeDtypeStruct((NC, NP, H), jnp.float32),
        mesh=_mesh,
        scratch_types=[
            pltpu.VMEM((CB, EC), jnp.int32),
            pltpu.VMEM((CB, EC), jnp.int32),
            pltpu.VMEM((CB, EC), jnp.int32),
            pltpu.VMEM((CB, EC), jnp.int32),
            pltpu.VMEM((EC, H), jnp.float32),
            pltpu.VMEM((EC, H), jnp.float32),
            pltpu.VMEM_SHARED((NP, H), jnp.float32),
            pltpu.SemaphoreType.DMA,
            pltpu.SemaphoreType.DMA,
            pltpu.SemaphoreType.DMA,
            pltpu.SemaphoreType.DMA,
        ],
    )
    def k(u_hbm, src_hbm, dst_hbm, out_hbm, is0, is1, id0, id1, buf0, buf1,
          acc, isem0, isem1, gsem0, gsem1):
        c = lax.axis_index("c")
        s = lax.axis_index("s")
        srcc = src_hbm.at[c].at[s]
        dstc = dst_hbm.at[c].at[s]
        iss = (is0, is1)
        ids = (id0, id1)
        isems = (isem0, isem1)

        def load_block(b):
            sl = pl.ds(b * CB, CB)
            sem = isems[b % 2]
            pltpu.async_copy(srcc.at[sl], iss[b % 2], sem)
            pltpu.async_copy(dstc.at[sl], ids[b % 2], sem)

        def wait_block(b):
            sl = pl.ds(b * CB, CB)
            sem = isems[b % 2]
            pltpu.make_async_copy(srcc.at[sl], iss[b % 2], sem).wait()
            pltpu.make_async_copy(dstc.at[sl], ids[b % 2], sem).wait()

        load_block(0)

        zero16 = jnp.zeros((16,), jnp.float32)

        @pl.loop(0, EC)
        def _(i):
            @pl.loop(0, H, step=16)
            def _(jj):
                buf0[i, pl.ds(jj, 16)] = zero16

        base = pl.multiple_of(s * RPS, 8)

        @pl.loop(0, KF)
        def _(kk):
            pltpu.sync_copy(buf0, acc.at[pl.ds(base + kk * EC, EC)])

        pltpu.sync_copy(buf0.at[pl.ds(0, KT)],
                        acc.at[pl.ds(base + KF * EC, KT)])
        plsc.subcore_barrier()

        wait_block(0)
        load_block(1)
        pltpu.async_copy(u_hbm.at[is0.at[0]], buf0, gsem0)

        for b in range(NB):
            isb = iss[b % 2]
            idb = ids[b % 2]

            @pl.loop(0, CB - 2, step=2)
            def _(j):
                pltpu.async_copy(u_hbm.at[isb.at[j + 1]], buf1, gsem1)
                pltpu.make_async_copy(u_hbm.at[isb.at[j]], buf0, gsem0).wait()
                pltpu.sync_copy(buf0, acc.at[idb.at[j]], add=True)
                pltpu.async_copy(u_hbm.at[isb.at[j + 2]], buf0, gsem0)
                pltpu.make_async_copy(u_hbm.at[isb.at[j + 1]], buf1,
                                      gsem1).wait()
                pltpu.sync_copy(buf1, acc.at[idb.at[j + 1]], add=True)

            pltpu.async_copy(u_hbm.at[isb.at[CB - 1]], buf1, gsem1)
            pltpu.make_async_copy(u_hbm.at[isb.at[CB - 2]], buf0, gsem0).wait()
            pltpu.sync_copy(buf0, acc.at[idb.at[CB - 2]], add=True)
            if b + 1 < NB:
                wait_block(b + 1)
                pltpu.async_copy(u_hbm.at[iss[(b + 1) % 2].at[0]], buf0, gsem0)
            pltpu.make_async_copy(u_hbm.at[isb.at[CB - 1]], buf1, gsem1).wait()
            pltpu.sync_copy(buf1, acc.at[idb.at[CB - 1]], add=True)
            if b + 2 < NB:
                load_block(b + 2)

        plsc.subcore_barrier()
        pltpu.sync_copy(acc.at[pl.ds(base, RPS)],
                        out_hbm.at[c].at[pl.ds(base, RPS)])

    return k(u, srcm, dstm)


def _valid_mask():
    rows = lax.broadcasted_iota(jnp.int32, (NP, H), 0)
    return rows < N


def _tc_pre(x, W1, hist):
    def body(x_ref, w_ref, hist_ref, u_ref, dinv_ref):
        xw = jnp.dot(x_ref[...], w_ref[...], preferred_element_type=jnp.float32)
        cnt = hist_ref[0, :, 0:1] + hist_ref[1, :, 0:1]
        deg = jnp.broadcast_to(cnt + 1.0, (NP, H))
        dinv = jnp.where(_valid_mask(), lax.rsqrt(deg), 0.0)
        dinv_ref[...] = dinv
        u_ref[:N, :] = xw * dinv[:N, :]
        u_ref[N:, :] = jnp.zeros((NP - N, H), jnp.float32)

    return pl.pallas_call(
        body,
        out_shape=(jax.ShapeDtypeStruct((NP, H), jnp.float32),
                   jax.ShapeDtypeStruct((NP, H), jnp.float32)),
    )(x, W1, hist)


def _bn_relu(y_ref, u_ref, dinv, b_ref, g_ref, be_ref):
    z = dinv * (y_ref[0] + y_ref[1] + u_ref[...]) + b_ref[...][None, :]
    zm = jnp.where(_valid_mask(), z, 0.0)
    s1 = jnp.sum(zm, axis=0)
    s2 = jnp.sum(zm * zm, axis=0)
    m = s1 / N
    v = s2 / N - m * m
    scale = lax.rsqrt(v + 1e-5) * g_ref[...]
    return jnp.maximum((z - m[None, :]) * scale[None, :] + be_ref[...][None, :],
                       0.0)


def _tc_mid(y, u, dinv, b, g, be, Wn):
    def body(y_ref, u_ref, dinv_ref, b_ref, g_ref, be_ref, w_ref, out_ref):
        dinv = dinv_ref[...]
        hn = _bn_relu(y_ref, u_ref, dinv, b_ref, g_ref, be_ref)
        out_ref[...] = dinv * jnp.dot(hn, w_ref[...],
                                      preferred_element_type=jnp.float32)

    return pl.pallas_call(
        body,
        out_shape=jax.ShapeDtypeStruct((NP, H), jnp.float32),
    )(y, u, dinv, b, g, be, Wn)


def _tc_post(y, u, dinv, b, g, be, batch_p, Wout, bout):
    def body(y_ref, u_ref, dinv_ref, b_ref, g_ref, be_ref, batch_ref,
             wout_ref, bout_ref, out_ref):
        hn = _bn_relu(y_ref, u_ref, dinv_ref[...], b_ref, g_ref, be_ref)
        gids = lax.broadcasted_iota(jnp.int32, (G, NP), 0)
        onehot = (batch_ref[...][None, :] == gids).astype(jnp.float32)
        cnt = jnp.sum(onehot, axis=1)
        pooled = jnp.dot(onehot, hn, preferred_element_type=jnp.float32)
        pooled = pooled / jnp.maximum(cnt, 1.0)[:, None]
        out_ref[...] = (jnp.dot(pooled, wout_ref[...],
                                preferred_element_type=jnp.float32)
                        + bout_ref[...][None, :])

    return pl.pallas_call(
        body,
        out_shape=jax.ShapeDtypeStruct((G, T), jnp.float32),
    )(y, u, dinv, b, g, be, batch_p, Wout, bout)


def kernel(x, edge_index, batch, W1, b1, g1, be1, W2, b2, g2, be2,
           W3, b3, g3, be3, Wout, bout):
    src = edge_index[0]
    dst = edge_index[1]
    padv = N + jnp.arange(EPAD - E, dtype=jnp.int32) % (NP - N)
    srcm = jnp.concatenate([src, padv]).reshape(NC, NS, CPS, EC)
    dstm = jnp.concatenate([dst, padv]).reshape(NC, NS, CPS, EC)
    batch_p = jnp.concatenate([batch, jnp.full((NP - N,), G, jnp.int32)])

    hist = _sc_hist(dstm)
    u1, dinv = _tc_pre(x, W1, hist)
    y1 = _sc_scatter(u1, srcm, dstm)
    u2 = _tc_mid(y1, u1, dinv, b1, g1, be1, W2)
    y2 = _sc_scatter(u2, srcm, dstm)
    u3 = _tc_mid(y2, u2, dinv, b2, g2, be2, W3)
    y3 = _sc_scatter(u3, srcm, dstm)
    return _tc_post(y3, u3, dinv, b3, g3, be3, batch_p, Wout, bout)


# speedup vs baseline: 29.5436x; 1.0042x over previous
import functools

import jax
import jax.numpy as jnp
from jax import lax
from jax.experimental import pallas as pl
from jax.experimental.pallas import tpu as pltpu
from jax.experimental.pallas import tpu_sc as plsc

N = 10000
E = 320000
D = 128
H = 128
G = 32
T = 5

NC = 2
NS = 16
NP = 10112
RPS = NP // NS
EC = 128
CPS = 80
EPAD = NC * NS * CPS * EC
CB = 16
NB = CPS // CB
KF = RPS // EC
KT = RPS % EC

_mesh = plsc.VectorSubcoreMesh(core_axis_name="c", subcore_axis_name="s")


@jax.jit
def _sc_hist(dstm):
    @functools.partial(
        pl.kernel,
        out_type=jax.ShapeDtypeStruct((NC, NP, 16), jnp.float32),
        mesh=_mesh,
        scratch_types=[
            pltpu.VMEM((CPS, EC), jnp.int32),
            pltpu.VMEM((EC, 16), jnp.float32),
            pltpu.VMEM_SHARED((NP, 16), jnp.float32),
        ],
    )
    def k(dst_hbm, out_hbm, idx_v, buf, acc):
        c = lax.axis_index("c")
        s = lax.axis_index("s")
        pltpu.sync_copy(dst_hbm.at[c].at[s], idx_v)

        zero16 = jnp.zeros((16,), jnp.float32)

        @pl.loop(0, EC)
        def _(i):
            buf[i, pl.ds(0, 16)] = zero16

        base = pl.multiple_of(s * RPS, 8)

        @pl.loop(0, KF)
        def _(kk):
            pltpu.sync_copy(buf, acc.at[pl.ds(base + kk * EC, EC)])

        pltpu.sync_copy(buf.at[pl.ds(0, KT)],
                        acc.at[pl.ds(base + KF * EC, KT)])

        one16 = jnp.ones((16,), jnp.float32)

        @pl.loop(0, EC)
        def _(i):
            buf[i, pl.ds(0, 16)] = one16

        plsc.subcore_barrier()

        @pl.loop(0, CPS)
        def _(j):
            pltpu.sync_copy(buf, acc.at[idx_v.at[j]], add=True)

        plsc.subcore_barrier()
        pltpu.sync_copy(acc.at[pl.ds(base, RPS)],
                        out_hbm.at[c].at[pl.ds(base, RPS)])

    return k(dstm)


@jax.jit
def _sc_scatter(u, srcm, dstm):
    @functools.partial(
        pl.kernel,
        out_type=jax.ShapeDtypeStruct((NC, NP, H), jnp.float32),
        mesh=_mesh,
        scratch_types=[
            pltpu.VMEM((CB, EC), jnp.int32),
            pltpu.VMEM((CB, EC), jnp.int32),
            pltpu.VMEM((CB, EC), jnp.int32),
            pltpu.VMEM((CB, EC), jnp.int32),
            pltpu.VMEM((EC, H), jnp.float32),
            pltpu.VMEM((EC, H), jnp.float32),
            pltpu.VMEM_SHARED((NP, H), jnp.float32),
            pltpu.SemaphoreType.DMA,
            pltpu.SemaphoreType.DMA,
            pltpu.SemaphoreType.DMA,
            pltpu.SemaphoreType.DMA,
        ],
    )
    def k(u_hbm, src_hbm, dst_hbm, out_hbm, is0, is1, id0, id1, buf0, buf1,
          acc, isem0, isem1, gsem0, gsem1):
        c = lax.axis_index("c")
        s = lax.axis_index("s")
        srcc = src_hbm.at[c].at[s]
        dstc = dst_hbm.at[c].at[s]
        iss = (is0, is1)
        ids = (id0, id1)
        isems = (isem0, isem1)

        def load_block(b):
            sl = pl.ds(b * CB, CB)
            sem = isems[b % 2]
            pltpu.async_copy(srcc.at[sl], iss[b % 2], sem)
            pltpu.async_copy(dstc.at[sl], ids[b % 2], sem)

        def wait_block(b):
            sl = pl.ds(b * CB, CB)
            sem = isems[b % 2]
            pltpu.make_async_copy(srcc.at[sl], iss[b % 2], sem).wait()
            pltpu.make_async_copy(dstc.at[sl], ids[b % 2], sem).wait()

        load_block(0)

        zero16 = jnp.zeros((16,), jnp.float32)

        @pl.loop(0, EC)
        def _(i):
            @pl.loop(0, H, step=16)
            def _(jj):
                buf0[i, pl.ds(jj, 16)] = zero16

        base = pl.multiple_of(s * RPS, 8)

        @pl.loop(0, KF)
        def _(kk):
            pltpu.sync_copy(buf0, acc.at[pl.ds(base + kk * EC, EC)])

        pltpu.sync_copy(buf0.at[pl.ds(0, KT)],
                        acc.at[pl.ds(base + KF * EC, KT)])
        plsc.subcore_barrier()

        wait_block(0)
        load_block(1)
        pltpu.async_copy(u_hbm.at[is0.at[0]], buf0, gsem0)

        for b in range(NB):
            isb = iss[b % 2]
            idb = ids[b % 2]
            bufs = (buf0, buf1)
            gsems = (gsem0, gsem1)

            for j in range(CB):
                bj = bufs[j % 2]
                gj = gsems[j % 2]
                bn = bufs[(j + 1) % 2]
                gn = gsems[(j + 1) % 2]
                if j + 1 < CB:
                    pltpu.async_copy(u_hbm.at[isb.at[j + 1]], bn, gn)
                elif b + 1 < NB:
                    wait_block(b + 1)
                    pltpu.async_copy(u_hbm.at[iss[(b + 1) % 2].at[0]], bn, gn)
                pltpu.make_async_copy(u_hbm.at[isb.at[j]], bj, gj).wait()
                pltpu.sync_copy(bj, acc.at[idb.at[j]], add=True)
            if b + 2 < NB:
                load_block(b + 2)

        plsc.subcore_barrier()
        pltpu.sync_copy(acc.at[pl.ds(base, RPS)],
                        out_hbm.at[c].at[pl.ds(base, RPS)])

    return k(u, srcm, dstm)


def _valid_mask():
    rows = lax.broadcasted_iota(jnp.int32, (NP, H), 0)
    return rows < N


def _tc_pre(x, W1, hist):
    def body(x_ref, w_ref, hist_ref, u_ref, dinv_ref):
        xw = jnp.dot(x_ref[...], w_ref[...], preferred_element_type=jnp.float32)
        cnt = hist_ref[0, :, 0:1] + hist_ref[1, :, 0:1]
        deg = jnp.broadcast_to(cnt + 1.0, (NP, H))
        dinv = jnp.where(_valid_mask(), lax.rsqrt(deg), 0.0)
        dinv_ref[...] = dinv
        u_ref[:N, :] = xw * dinv[:N, :]
        u_ref[N:, :] = jnp.zeros((NP - N, H), jnp.float32)

    return pl.pallas_call(
        body,
        out_shape=(jax.ShapeDtypeStruct((NP, H), jnp.float32),
                   jax.ShapeDtypeStruct((NP, H), jnp.float32)),
    )(x, W1, hist)


def _bn_relu(y_ref, u_ref, dinv, b_ref, g_ref, be_ref):
    z = dinv * (y_ref[0] + y_ref[1] + u_ref[...]) + b_ref[...][None, :]
    zm = jnp.where(_valid_mask(), z, 0.0)
    s1 = jnp.sum(zm, axis=0)
    s2 = jnp.sum(zm * zm, axis=0)
    m = s1 / N
    v = s2 / N - m * m
    scale = lax.rsqrt(v + 1e-5) * g_ref[...]
    return jnp.maximum((z - m[None, :]) * scale[None, :] + be_ref[...][None, :],
                       0.0)


def _tc_mid(y, u, dinv, b, g, be, Wn):
    def body(y_ref, u_ref, dinv_ref, b_ref, g_ref, be_ref, w_ref, out_ref):
        dinv = dinv_ref[...]
        hn = _bn_relu(y_ref, u_ref, dinv, b_ref, g_ref, be_ref)
        out_ref[...] = dinv * jnp.dot(hn, w_ref[...],
                                      preferred_element_type=jnp.float32)

    return pl.pallas_call(
        body,
        out_shape=jax.ShapeDtypeStruct((NP, H), jnp.float32),
    )(y, u, dinv, b, g, be, Wn)


def _tc_post(y, u, dinv, b, g, be, batch_p, Wout, bout):
    def body(y_ref, u_ref, dinv_ref, b_ref, g_ref, be_ref, batch_ref,
             wout_ref, bout_ref, out_ref):
        hn = _bn_relu(y_ref, u_ref, dinv_ref[...], b_ref, g_ref, be_ref)
        gids = lax.broadcasted_iota(jnp.int32, (G, NP), 0)
        onehot = (batch_ref[...][None, :] == gids).astype(jnp.float32)
        cnt = jnp.sum(onehot, axis=1)
        pooled = jnp.dot(onehot, hn, preferred_element_type=jnp.float32)
        pooled = pooled / jnp.maximum(cnt, 1.0)[:, None]
        out_ref[...] = (jnp.dot(pooled, wout_ref[...],
                                preferred_element_type=jnp.float32)
                        + bout_ref[...][None, :])

    return pl.pallas_call(
        body,
        out_shape=jax.ShapeDtypeStruct((G, T), jnp.float32),
    )(y, u, dinv, b, g, be, batch_p, Wout, bout)


def kernel(x, edge_index, batch, W1, b1, g1, be1, W2, b2, g2, be2,
           W3, b3, g3, be3, Wout, bout):
    src = edge_index[0]
    dst = edge_index[1]
    padv = N + jnp.arange(EPAD - E, dtype=jnp.int32) % (NP - N)
    srcm = jnp.concatenate([src, padv]).reshape(NC, NS, CPS, EC)
    dstm = jnp.concatenate([dst, padv]).reshape(NC, NS, CPS, EC)
    batch_p = jnp.concatenate([batch, jnp.full((NP - N,), G, jnp.int32)])

    hist = _sc_hist(dstm)
    u1, dinv = _tc_pre(x, W1, hist)
    y1 = _sc_scatter(u1, srcm, dstm)
    u2 = _tc_mid(y1, u1, dinv, b1, g1, be1, W2)
    y2 = _sc_scatter(u2, srcm, dstm)
    u3 = _tc_mid(y2, u2, dinv, b2, g2, be2, W3)
    y3 = _sc_scatter(u3, srcm, dstm)
    return _tc_post(y3, u3, dinv, b3, g3, be3, batch_p, Wout, bout)
